# Initial kernel scaffold; baseline (speedup 1.0000x reference)
import jax
import jax.numpy as jnp
from jax.experimental import pallas as pl


def kernel(x, ln_scale, ln_bias, cp_W, cp_b, gl_W, gl_b, Wg, Wu, Wd):
    raise NotImplementedError("write your pallas kernel here")


# speedup vs baseline: 2.6459x; 2.6459x over previous
import jax
import jax.numpy as jnp
from jax import lax
from jax.experimental import pallas as pl
from jax.experimental.pallas import tpu as pltpu

_PS = 4
_CAP = 2


def _moe_body(K, P, T, D, E,
              p_ref, lns_ref, lnb_ref, cpW_ref, cpb_ref, glW_ref, glb_ref,
              Wg_ref, Wu_ref, Wd_ref, out_ref, posm_ref, wtn_ref):
    e = pl.program_id(1)

    @pl.when(e == 0)
    def _gating():
        p = p_ref[0]
        pm = jnp.mean(p, axis=0)
        pv = jnp.mean((p - pm[None, :, :]) ** 2, axis=0)
        mu = jnp.mean(pm, axis=1, keepdims=True)
        var = jnp.mean((pm - mu) ** 2, axis=1, keepdims=True)
        ln = (pm - mu) / jnp.sqrt(var + 1e-5) * lns_ref[...] + lnb_ref[...]
        cpv = jnp.dot(pv, cpW_ref[...],
                      preferred_element_type=jnp.float32) + cpb_ref[...]
        logits = (jnp.dot(ln, glW_ref[0:D, :], preferred_element_type=jnp.float32)
                  + jnp.dot(cpv, glW_ref[D:2 * D, :], preferred_element_type=jnp.float32)
                  + glb_ref[...])
        mx = jnp.max(logits, axis=1, keepdims=True)
        ex = jnp.exp(logits - mx)
        aff = ex / jnp.sum(ex, axis=1, keepdims=True)
        affT = jnp.transpose(aff)
        a_t = affT[:, :, None]
        a_s = affT[:, None, :]
        i_t = lax.broadcasted_iota(jnp.int32, (E, T, T), 1)
        i_s = lax.broadcasted_iota(jnp.int32, (E, T, T), 2)
        beats = (a_s > a_t) | ((a_s == a_t) & (i_s < i_t))
        rank = jnp.sum(beats.astype(jnp.float32), axis=2)
        maskf = (rank < K).astype(jnp.float32)
        ii = lax.broadcasted_iota(jnp.int32, (T, T), 0)
        jj = lax.broadcasted_iota(jnp.int32, (T, T), 1)
        lt = (ii < jj).astype(jnp.float32)
        pos = jnp.dot(maskf, lt, preferred_element_type=jnp.float32)
        posm_ref[...] = jnp.where(maskf > 0, pos, -1.0).astype(jnp.int32)
        tot = jnp.sum(maskf * affT, axis=0, keepdims=True)
        wtn_ref[...] = maskf * affT / jnp.maximum(tot, 1e-8)
        out_ref[...] = jnp.zeros_like(out_ref)

    posm_row = posm_ref[pl.ds(e, 1), :]
    wt_row = wtn_ref[pl.ds(e, 1), :]
    jrow = lax.broadcasted_iota(jnp.int32, (K, T), 0)
    G = (jrow == posm_row).astype(jnp.float32)
    GwT = jnp.transpose(G * wt_row)

    psel = jnp.concatenate(
        [jnp.dot(G, p_ref[0, j], preferred_element_type=jnp.float32)
         for j in range(P)], axis=0)
    hg = jnp.dot(psel, Wg_ref[0], preferred_element_type=jnp.float32)
    hu = jnp.dot(psel, Wu_ref[0], preferred_element_type=jnp.float32)
    h = hg * lax.logistic(hg) * hu
    proc = jnp.dot(h, Wd_ref[0], preferred_element_type=jnp.float32)
    for j in range(P):
        c = jnp.dot(GwT, proc[j * K:(j + 1) * K, :],
                    preferred_element_type=jnp.float32)
        out_ref[0, j] += c


def kernel(x, ln_scale, ln_bias, cp_W, cp_b, gl_W, gl_b, Wg, Wu, Wd):
    B, S, D = x.shape
    E = gl_W.shape[1]
    HW = int(round(S ** 0.5))
    Th = HW // _PS
    T = Th * Th
    P = _PS * _PS
    K = max(1, int(T / E * _CAP))

    x2d = x.reshape(B, HW, HW, D)
    p_tok = (x2d.reshape(B, Th, _PS, Th, _PS, D)
             .transpose(0, 2, 4, 1, 3, 5)
             .reshape(B, P, T, D))

    def body(*refs):
        _moe_body(K, P, T, D, E, *refs)

    out_tok = pl.pallas_call(
        body,
        grid=(B, E),
        in_specs=[
            pl.BlockSpec((1, P, T, D), lambda b, e: (b, 0, 0, 0)),
            pl.BlockSpec((1, D), lambda b, e: (0, 0)),
            pl.BlockSpec((1, D), lambda b, e: (0, 0)),
            pl.BlockSpec((D, D), lambda b, e: (0, 0)),
            pl.BlockSpec((1, D), lambda b, e: (0, 0)),
            pl.BlockSpec((2 * D, E), lambda b, e: (0, 0)),
            pl.BlockSpec((1, E), lambda b, e: (0, 0)),
            pl.BlockSpec((1, D, Wg.shape[2]), lambda b, e: (e, 0, 0)),
            pl.BlockSpec((1, D, Wu.shape[2]), lambda b, e: (e, 0, 0)),
            pl.BlockSpec((1, Wd.shape[1], D), lambda b, e: (e, 0, 0)),
        ],
        out_specs=pl.BlockSpec((1, P, T, D), lambda b, e: (b, 0, 0, 0)),
        out_shape=jax.ShapeDtypeStruct((B, P, T, D), jnp.float32),
        scratch_shapes=[
            pltpu.VMEM((E, T), jnp.int32),
            pltpu.VMEM((E, T), jnp.float32),
        ],
    )(p_tok, ln_scale.reshape(1, D), ln_bias.reshape(1, D), cp_W,
      cp_b.reshape(1, D), gl_W, gl_b.reshape(1, E), Wg, Wu, Wd)

    out2d = (out_tok.reshape(B, _PS, _PS, Th, Th, D)
             .transpose(0, 3, 1, 4, 2, 5)
             .reshape(B, HW, HW, D))
    return out2d.reshape(B, S, D)
